# Initial kernel scaffold; baseline (speedup 1.0000x reference)
import jax
import jax.numpy as jnp
from jax.experimental import pallas as pl


def kernel(input, target):
    raise NotImplementedError("write your pallas kernel here")


# speedup vs baseline: 1.0749x; 1.0749x over previous
import functools

import jax
import jax.numpy as jnp
from jax import lax
from jax.experimental import pallas as pl
from jax.experimental.pallas import tpu as pltpu
from jax.experimental.pallas import tpu_sc as plsc

_H = 512
_W = 512
_NCMP = 20000
_NC = 2
_NS = 16
_NW = _NC * _NS
_L = 16
_NPAD = 20480
_PER_W = _NPAD // _NW
_NV = _PER_W // _L
_IROWS = 2 * _PER_W // 128
_HROWS = _IROWS // 2


def _whdr_partials(img_flat, tgt):
    mesh = plsc.VectorSubcoreMesh(core_axis_name="c", subcore_axis_name="s")

    @functools.partial(
        pl.kernel,
        mesh=mesh,
        out_type=jax.ShapeDtypeStruct((_NC, 2, _L), jnp.float32),
        scratch_types=[
            pltpu.VMEM((6, _PER_W), jnp.float32),
            pltpu.VMEM((_IROWS, 128), jnp.int32),
            pltpu.VMEM((_IROWS, 128), jnp.float32),
            pltpu.VMEM((_L,), jnp.int32),
            pltpu.VMEM((_L,), jnp.float32),
            pltpu.VMEM((_L,), jnp.float32),
            pltpu.VMEM_SHARED((_L,), jnp.float32),
            pltpu.VMEM_SHARED((_L,), jnp.float32),
            pltpu.SemaphoreType.DMA,
        ],
    )
    def whdr_kernel(img_hbm, tgt_hbm, out_hbm, tv, idxv, pixv, iotav,
                    numv, denv, sh_num, sh_den, sem):
        c = lax.axis_index("c")
        s = lax.axis_index("s")
        wid = s * _NC + c
        base = wid * _PER_W

        zeros = jnp.zeros((_L,), jnp.float32)
        numv[...] = zeros

        @pl.when(s == 0)
        def _init_shared():
            pltpu.sync_copy(numv, sh_num)
            pltpu.sync_copy(numv, sh_den)

        for r in range(6):
            pltpu.sync_copy(tgt_hbm.at[r, pl.ds(base, _PER_W)], tv.at[r])

        for v in range(_NV):
            sl = pl.ds(v * _L, _L)
            row, col = v // 8, (v % 8) * _L
            x1 = (tv[2, sl] * _W).astype(jnp.int32)
            y1 = (tv[3, sl] * _H).astype(jnp.int32)
            x2 = (tv[4, sl] * _W).astype(jnp.int32)
            y2 = (tv[5, sl] * _H).astype(jnp.int32)
            idxv[row, pl.ds(col, _L)] = y1 * _W + x1
            idxv[_HROWS + row, pl.ds(col, _L)] = y2 * _W + x2

        copies = [pltpu.async_copy(img_hbm.at[idxv.at[j]], pixv.at[j], sem)
                  for j in range(_IROWS)]
        for cp in copies:
            cp.wait()

        hi = 1.0 + 0.12
        lo = 1.0 / (1.0 + 0.12)
        num = zeros
        den = zeros
        for v in range(_NV):
            sl = pl.ds(v * _L, _L)
            row, col = v // 8, (v % 8) * _L
            r1 = pixv[row, pl.ds(col, _L)]
            r2 = pixv[_HROWS + row, pl.ds(col, _L)]
            ratio = r1 / (r2 + 1e-07)
            pred = jnp.where(ratio > hi, 2.0, jnp.where(ratio < lo, 1.0, 0.0))
            wt = tv[0, sl]
            lab = tv[1, sl]
            num = num + jnp.where(lab != pred, wt, zeros)
            den = den + wt
        numv[...] = num
        denv[...] = den
        iotav[...] = lax.iota(jnp.int32, _L)

        plsc.subcore_barrier()
        pltpu.sync_copy(numv, sh_num.at[iotav], add=True)
        pltpu.sync_copy(denv, sh_den.at[iotav], add=True)
        plsc.subcore_barrier()

        @pl.when(s == 0)
        def _finish():
            pltpu.sync_copy(sh_num, out_hbm.at[c, 0])
            pltpu.sync_copy(sh_den, out_hbm.at[c, 1])

    return whdr_kernel(img_flat, tgt)


def kernel(input, target):
    img_flat = input.reshape(_H * _W)
    tpad = jnp.pad(target[0], ((0, _NPAD - _NCMP), (0, 0)))
    parts = _whdr_partials(img_flat, tpad.T)
    num = jnp.sum(parts[:, 0, :])
    den = jnp.sum(parts[:, 1, :])
    return (num / den).reshape(1)
